# baseline (speedup 1.0000x reference)
import math

import jax
import jax.numpy as jnp
from jax import lax
from jax.experimental import pallas as pl
from jax.experimental.pallas import tpu as pltpu
from jax.experimental.pallas import tpu_sc as plsc

VOCAB = 1000000
D_MODEL = 64
SCALE = math.sqrt(D_MODEL)

NUM_CORES = 2
NUM_SUBCORES = 16
NW = NUM_CORES * NUM_SUBCORES
LANES = 16

NJ = 200
NS = 4096
SBLK = NS // NW


def _emb_body(xT_hbm, table_hbm, out_hbm,
              idx0, idx1, rows0, rows1, st0, st1,
              gsem0, gsem1, wsem0, wsem1):
    wid = lax.axis_index("s") * NUM_CORES + lax.axis_index("c")
    s0 = wid * SBLK

    idx = (idx0, idx1)
    rows = (rows0, rows1)
    st = (st0, st1)
    gsem = (gsem0, gsem1)
    wsem = (wsem0, wsem1)

    def transpose_scale(rbuf, sbuf):
        @pl.loop(0, D_MODEL)
        def _d(d):
            cols = jnp.full((16,), d, dtype=jnp.int32)
            for k in range(SBLK // LANES):
                rids = lax.iota(jnp.int32, 16) + (k * LANES)
                v = plsc.load_gather(rbuf, [rids, cols])
                sbuf[d, pl.ds(k * LANES, LANES)] = v * SCALE

    pltpu.sync_copy(xT_hbm.at[0, pl.ds(s0, SBLK)], idx[0])
    pltpu.async_copy(table_hbm.at[idx[0]], rows[0], gsem[0])

    @pl.loop(0, NJ, step=2)
    def _pair(g):
        for ph in range(2):
            cur, nxt = ph, 1 - ph
            jj = g + ph

            @pl.when(jj + 1 < NJ)
            def _prefetch():
                pltpu.sync_copy(xT_hbm.at[jj + 1, pl.ds(s0, SBLK)], idx[nxt])

                @pl.when(jj >= 1)
                def _drain_prev():
                    pltpu.make_async_copy(
                        st[nxt],
                        out_hbm.at[jj - 1, :, pl.ds(s0, SBLK)],
                        wsem[nxt]).wait()

                pltpu.async_copy(table_hbm.at[idx[nxt]], rows[nxt], gsem[nxt])

            pltpu.make_async_copy(
                table_hbm.at[idx[cur]], rows[cur], gsem[cur]).wait()
            transpose_scale(rows[cur], st[cur])
            pltpu.async_copy(
                st[cur], out_hbm.at[jj, :, pl.ds(s0, SBLK)], wsem[cur])

    pltpu.make_async_copy(
        st[0], out_hbm.at[NJ - 2, :, pl.ds(s0, SBLK)], wsem[0]).wait()
    pltpu.make_async_copy(
        st[1], out_hbm.at[NJ - 1, :, pl.ds(s0, SBLK)], wsem[1]).wait()


@jax.jit
def _emb_call(xT, table):
    mesh = plsc.VectorSubcoreMesh(core_axis_name="c", subcore_axis_name="s")
    return pl.kernel(
        _emb_body,
        out_type=jax.ShapeDtypeStruct((NJ, D_MODEL, NS), jnp.float32),
        mesh=mesh,
        scratch_types=[
            pltpu.VMEM((SBLK,), jnp.int32),
            pltpu.VMEM((SBLK,), jnp.int32),
            pltpu.VMEM((SBLK, D_MODEL), jnp.float32),
            pltpu.VMEM((SBLK, D_MODEL), jnp.float32),
            pltpu.VMEM((D_MODEL, SBLK), jnp.float32),
            pltpu.VMEM((D_MODEL, SBLK), jnp.float32),
            pltpu.SemaphoreType.DMA,
            pltpu.SemaphoreType.DMA,
            pltpu.SemaphoreType.DMA,
            pltpu.SemaphoreType.DMA,
        ],
        compiler_params=pltpu.CompilerParams(
            use_tc_tiling_on_sc=False, needs_layout_passes=False),
    )(xT, table)


def kernel(x, table):
    out3 = _emb_call(x.T.astype(jnp.int32), table)
    return out3.transpose(2, 0, 1)


# speedup vs baseline: 1.3840x; 1.3840x over previous
import math

import jax
import jax.numpy as jnp
from jax import lax
from jax.experimental import pallas as pl
from jax.experimental.pallas import tpu as pltpu
from jax.experimental.pallas import tpu_sc as plsc

VOCAB = 1000000
D_MODEL = 64
SCALE = math.sqrt(D_MODEL)

NUM_CORES = 2
NUM_SUBCORES = 16
NW = NUM_CORES * NUM_SUBCORES
LANES = 16

NJ = 200
NS = 4096
SBLK = NS // NW


def _emb_body(xT_hbm, table_hbm, out_hbm,
              idx0, idx1, rows0, rows1, st0, st1,
              gsem0, gsem1, wsem0, wsem1):
    wid = lax.axis_index("s") * NUM_CORES + lax.axis_index("c")
    s0 = wid * SBLK

    idx = (idx0, idx1)
    rows = (rows0, rows1)
    st = (st0, st1)
    gsem = (gsem0, gsem1)
    wsem = (wsem0, wsem1)

    def transpose_scale(rbuf, sbuf):
        @plsc.parallel_loop(0, D_MODEL, unroll=8)
        def _d(d):
            cols = jnp.full((16,), d, dtype=jnp.int32)
            for k in range(SBLK // LANES):
                rids = lax.iota(jnp.int32, 16) + (k * LANES)
                v = plsc.load_gather(rbuf, [rids, cols])
                sbuf[d, pl.ds(k * LANES, LANES)] = v * SCALE

    pltpu.sync_copy(xT_hbm.at[0, pl.ds(s0, SBLK)], idx[0])
    pltpu.async_copy(table_hbm.at[idx[0]], rows[0], gsem[0])

    @pl.loop(0, NJ, step=2)
    def _pair(g):
        for ph in range(2):
            cur, nxt = ph, 1 - ph
            jj = g + ph

            @pl.when(jj + 1 < NJ)
            def _prefetch():
                pltpu.sync_copy(xT_hbm.at[jj + 1, pl.ds(s0, SBLK)], idx[nxt])

                @pl.when(jj >= 1)
                def _drain_prev():
                    pltpu.make_async_copy(
                        st[nxt],
                        out_hbm.at[jj - 1, :, pl.ds(s0, SBLK)],
                        wsem[nxt]).wait()

                pltpu.async_copy(table_hbm.at[idx[nxt]], rows[nxt], gsem[nxt])

            pltpu.make_async_copy(
                table_hbm.at[idx[cur]], rows[cur], gsem[cur]).wait()
            transpose_scale(rows[cur], st[cur])
            pltpu.async_copy(
                st[cur], out_hbm.at[jj, :, pl.ds(s0, SBLK)], wsem[cur])

    pltpu.make_async_copy(
        st[0], out_hbm.at[NJ - 2, :, pl.ds(s0, SBLK)], wsem[0]).wait()
    pltpu.make_async_copy(
        st[1], out_hbm.at[NJ - 1, :, pl.ds(s0, SBLK)], wsem[1]).wait()


@jax.jit
def _emb_call(xT, table):
    mesh = plsc.VectorSubcoreMesh(core_axis_name="c", subcore_axis_name="s")
    return pl.kernel(
        _emb_body,
        out_type=jax.ShapeDtypeStruct((NJ, D_MODEL, NS), jnp.float32),
        mesh=mesh,
        scratch_types=[
            pltpu.VMEM((SBLK,), jnp.int32),
            pltpu.VMEM((SBLK,), jnp.int32),
            pltpu.VMEM((SBLK, D_MODEL), jnp.float32),
            pltpu.VMEM((SBLK, D_MODEL), jnp.float32),
            pltpu.VMEM((D_MODEL, SBLK), jnp.float32),
            pltpu.VMEM((D_MODEL, SBLK), jnp.float32),
            pltpu.SemaphoreType.DMA,
            pltpu.SemaphoreType.DMA,
            pltpu.SemaphoreType.DMA,
            pltpu.SemaphoreType.DMA,
        ],
        compiler_params=pltpu.CompilerParams(
            use_tc_tiling_on_sc=False, needs_layout_passes=False),
    )(xT, table)


def kernel(x, table):
    out3 = _emb_call(x.T.astype(jnp.int32), table)
    return out3.transpose(2, 0, 1)


# speedup vs baseline: 1.4558x; 1.0519x over previous
import math

import jax
import jax.numpy as jnp
from jax import lax
from jax.experimental import pallas as pl
from jax.experimental.pallas import tpu as pltpu
from jax.experimental.pallas import tpu_sc as plsc

VOCAB = 1000000
D_MODEL = 64
SCALE = math.sqrt(D_MODEL)

NUM_CORES = 2
NUM_SUBCORES = 16
NW = NUM_CORES * NUM_SUBCORES
LANES = 16

NJ = 200
NS = 4096
SBLK = NS // NW


NSLOT = 8
NST = 2


def _emb_body(xT_hbm, table_hbm, out_hbm, idx_v, rows, st, gsem, wsem):
    wid = lax.axis_index("s") * NUM_CORES + lax.axis_index("c")
    s0 = wid * SBLK

    def fire_gather(j, slot):
        pltpu.async_copy(table_hbm.at[idx_v.at[j]], rows.at[slot],
                         gsem.at[slot])

    def wait_gather(j, slot):
        pltpu.make_async_copy(table_hbm.at[idx_v.at[j]], rows.at[slot],
                              gsem.at[slot]).wait()

    def fire_write(p, j):
        pltpu.async_copy(st.at[p], out_hbm.at[j, :, pl.ds(s0, SBLK)],
                         wsem.at[p])

    def wait_write(p, j):
        pltpu.make_async_copy(st.at[p], out_hbm.at[j, :, pl.ds(s0, SBLK)],
                              wsem.at[p]).wait()

    pltpu.sync_copy(xT_hbm.at[:, pl.ds(s0, SBLK)], idx_v)
    for jj in range(NSLOT):
        fire_gather(jj, jj)

    @pl.loop(0, NJ)
    def _j(j):
        slot = lax.rem(j, NSLOT)
        p = lax.rem(j, NST)
        wait_gather(j, slot)

        @pl.when(j >= NST)
        def _drain():
            wait_write(p, j - NST)

        slot_v = jnp.full((16,), slot, dtype=jnp.int32)

        @plsc.parallel_loop(0, D_MODEL, unroll=8)
        def _d(d):
            cols = jnp.full((16,), d, dtype=jnp.int32)
            for k in range(SBLK // LANES):
                rids = lax.iota(jnp.int32, 16) + (k * LANES)
                v = plsc.load_gather(rows, [slot_v, rids, cols])
                st[p, d, pl.ds(k * LANES, LANES)] = v * SCALE

        fire_write(p, j)

        @pl.when(j + NSLOT < NJ)
        def _refill():
            fire_gather(j + NSLOT, slot)

    wait_write(0, NJ - 2)
    wait_write(1, NJ - 1)


@jax.jit
def _emb_call(xT, table):
    mesh = plsc.VectorSubcoreMesh(core_axis_name="c", subcore_axis_name="s")
    return pl.kernel(
        _emb_body,
        out_type=jax.ShapeDtypeStruct((NJ, D_MODEL, NS), jnp.float32),
        mesh=mesh,
        scratch_types=[
            pltpu.VMEM((NJ, SBLK), jnp.int32),
            pltpu.VMEM((NSLOT, SBLK, D_MODEL), jnp.float32),
            pltpu.VMEM((NST, D_MODEL, SBLK), jnp.float32),
            pltpu.SemaphoreType.DMA((NSLOT,)),
            pltpu.SemaphoreType.DMA((NST,)),
        ],
        compiler_params=pltpu.CompilerParams(
            use_tc_tiling_on_sc=False, needs_layout_passes=False),
    )(xT, table)


def kernel(x, table):
    out3 = _emb_call(x.T.astype(jnp.int32), table)
    return out3.transpose(2, 0, 1)


# speedup vs baseline: 1.7767x; 1.2204x over previous
import math

import jax
import jax.numpy as jnp
from jax import lax
from jax.experimental import pallas as pl
from jax.experimental.pallas import tpu as pltpu
from jax.experimental.pallas import tpu_sc as plsc

VOCAB = 1000000
D_MODEL = 64
SCALE = math.sqrt(D_MODEL)

NUM_CORES = 2
NUM_SUBCORES = 16
NW = NUM_CORES * NUM_SUBCORES
LANES = 16

NJ = 200
NS = 4096
SBLK = NS // NW
PAIRW = 2 * D_MODEL

NSLOT = 4
NST = 2


def _emb_body(xT_hbm, tab2_hbm, out_hbm, idx_v, idx2, rows, st, gsem, wsem):
    wid = lax.axis_index("s") * NUM_CORES + lax.axis_index("c")
    s0 = wid * SBLK

    def prep_and_fire(j, slot):
        for k in range(SBLK // LANES):
            sl = pl.ds(k * LANES, LANES)
            idx2[slot, sl] = lax.shift_right_logical(idx_v[j, sl], 1)
        pltpu.async_copy(tab2_hbm.at[idx2.at[slot]], rows.at[slot],
                         gsem.at[slot])

    def wait_gather(slot):
        pltpu.make_async_copy(tab2_hbm.at[idx2.at[slot]], rows.at[slot],
                              gsem.at[slot]).wait()

    def fire_write(p, j):
        pltpu.async_copy(st.at[p], out_hbm.at[j, :, pl.ds(s0, SBLK)],
                         wsem.at[p])

    def wait_write(p, j):
        pltpu.make_async_copy(st.at[p], out_hbm.at[j, :, pl.ds(s0, SBLK)],
                              wsem.at[p]).wait()

    pltpu.sync_copy(xT_hbm.at[:, pl.ds(s0, SBLK)], idx_v)
    for jj in range(NSLOT):
        prep_and_fire(jj, jj)

    @pl.loop(0, NJ)
    def _j(j):
        slot = lax.rem(j, NSLOT)
        p = lax.rem(j, NST)
        wait_gather(slot)

        @pl.when(j >= NST)
        def _drain():
            wait_write(p, j - NST)

        slot_v = jnp.full((16,), slot, dtype=jnp.int32)
        pars = [
            lax.shift_left(
                jnp.bitwise_and(idx_v[j, pl.ds(k * LANES, LANES)], 1), 6)
            for k in range(SBLK // LANES)
        ]

        @plsc.parallel_loop(0, D_MODEL, unroll=8)
        def _d(d):
            dv = jnp.full((16,), d, dtype=jnp.int32)
            for k in range(SBLK // LANES):
                rids = lax.iota(jnp.int32, 16) + (k * LANES)
                v = plsc.load_gather(rows, [slot_v, rids, pars[k] + dv])
                st[p, d, pl.ds(k * LANES, LANES)] = v * SCALE

        fire_write(p, j)

        @pl.when(j + NSLOT < NJ)
        def _refill():
            prep_and_fire(j + NSLOT, slot)

    wait_write(0, NJ - 2)
    wait_write(1, NJ - 1)


@jax.jit
def _emb_call(xT, tab2):
    mesh = plsc.VectorSubcoreMesh(core_axis_name="c", subcore_axis_name="s")
    return pl.kernel(
        _emb_body,
        out_type=jax.ShapeDtypeStruct((NJ, D_MODEL, NS), jnp.float32),
        mesh=mesh,
        scratch_types=[
            pltpu.VMEM((NJ, SBLK), jnp.int32),
            pltpu.VMEM((NSLOT, SBLK), jnp.int32),
            pltpu.VMEM((NSLOT, SBLK, PAIRW), jnp.float32),
            pltpu.VMEM((NST, D_MODEL, SBLK), jnp.float32),
            pltpu.SemaphoreType.DMA((NSLOT,)),
            pltpu.SemaphoreType.DMA((NST,)),
        ],
        compiler_params=pltpu.CompilerParams(
            use_tc_tiling_on_sc=True, needs_layout_passes=False),
    )(xT, tab2)


def kernel(x, table):
    out3 = _emb_call(x.T.astype(jnp.int32),
                     table.reshape(VOCAB // 2, PAIRW))
    return out3.transpose(2, 0, 1)
